# baseline (speedup 1.0000x reference)
import functools

import jax
import jax.numpy as jnp
from jax import lax
from jax.experimental import pallas as pl
from jax.experimental.pallas import tpu as pltpu
from jax.experimental.pallas import tpu_sc as plsc

P = 128
NROW = 10
NPAD = 16
B = 16384
BLK = 128


def _table_kernel(e0_ref, e1_ref, w1_ref, b1_ref, w2_ref, b2_ref, t_ref):
    a0 = jnp.dot(e0_ref[...], w1_ref[0:P, :], preferred_element_type=jnp.float32)
    a1 = jnp.dot(e1_ref[...], w1_ref[P:2 * P, :], preferred_element_type=jnp.float32)
    a1p = jnp.concatenate([a1, jnp.zeros((NPAD - NROW, P), jnp.float32)], axis=0)
    h = a0[:, None, :] + a1p[None, :, :] + b1_ref[...][None, :, :]
    h = jnp.maximum(h, 0.0)
    t = jnp.sum(h.reshape(NROW * NPAD, P) * w2_ref[...], axis=1) + b2_ref[0, 0]
    t_ref[...] = t


def _build_table(emb0, emb1, W1, b1, W2, b2):
    return pl.pallas_call(
        _table_kernel,
        out_shape=jax.ShapeDtypeStruct((NROW * NPAD,), jnp.float32),
    )(emb0, emb1, W1, b1.reshape(1, P), W2.T, b2.reshape(1, 1))


def _make_sc_gather():
    info = plsc.get_sparse_core_info()
    ncores = 1
    nw = ncores * info.num_subcores
    b_per_w = B // nw
    k_per_w = b_per_w // BLK

    mesh = plsc.VectorSubcoreMesh(core_axis_name="c", subcore_axis_name="s", num_cores=ncores)

    @functools.partial(
        pl.kernel,
        mesh=mesh,
        out_type=jax.ShapeDtypeStruct((B,), jnp.float32),
        compiler_params=pltpu.CompilerParams(
            needs_layout_passes=False, use_tc_tiling_on_sc=False
        ),
        scratch_types=[
            pltpu.VMEM((k_per_w, 2, BLK), jnp.int32),
            pltpu.VMEM((NROW * NPAD,), jnp.float32),
            pltpu.VMEM((b_per_w,), jnp.float32),
            pltpu.SemaphoreType.DMA,
            pltpu.SemaphoreType.DMA,
            pltpu.SemaphoreType.DMA,
        ],
    )
    def gather_k(xb_hbm, t_hbm, out_hbm, x_v, t_v, o_v, sem_x, sem_t, sem_o):
        wid = lax.axis_index("s") * ncores + lax.axis_index("c")
        base = wid * k_per_w
        cp_x = pltpu.async_copy(xb_hbm.at[pl.ds(base, k_per_w)], x_v, sem_x)
        cp_t = pltpu.async_copy(t_hbm, t_v, sem_t)
        cp_x.wait()
        cp_t.wait()
        out_cps = []
        for blk in range(k_per_w):
            for s in range(BLK // 16):
                x0 = x_v[blk, 0, pl.ds(s * 16, 16)]
                x1 = x_v[blk, 1, pl.ds(s * 16, 16)]
                vals = plsc.load_gather(t_v, [x0 * NPAD + x1])
                o_v[pl.ds(blk * BLK + s * 16, 16)] = vals
            out_cps.append(pltpu.async_copy(
                o_v.at[pl.ds(blk * BLK, BLK)],
                out_hbm.at[pl.ds(wid * b_per_w + blk * BLK, BLK)],
                sem_o,
            ))
        for cp in out_cps:
            cp.wait()

    return gather_k


def kernel(x, emb0, emb1, W1, b1, W2, b2):
    t = _build_table(emb0, emb1, W1, b1, W2, b2)
    xb = x.reshape(B // BLK, BLK, 2).transpose(0, 2, 1)
    out = _make_sc_gather()(xb, t)
    return out.reshape(B, 1)


# speedup vs baseline: 1.0055x; 1.0055x over previous
import functools

import jax
import jax.numpy as jnp
from jax import lax
from jax.experimental import pallas as pl
from jax.experimental.pallas import tpu as pltpu
from jax.experimental.pallas import tpu_sc as plsc

P = 128
NROW = 10
NPAD = 16
B = 16384
BLK = 128


def _table_kernel(e0_ref, e1_ref, w1_ref, b1_ref, w2_ref, b2_ref, t_ref):
    a0 = jnp.dot(e0_ref[...], w1_ref[0:P, :], preferred_element_type=jnp.float32,
                 precision=lax.Precision.HIGHEST)
    a1 = jnp.dot(e1_ref[...], w1_ref[P:2 * P, :], preferred_element_type=jnp.float32,
                 precision=lax.Precision.HIGHEST)
    a1p = jnp.concatenate([a1, jnp.zeros((NPAD - NROW, P), jnp.float32)], axis=0)
    h = a0[:, None, :] + a1p[None, :, :] + b1_ref[...][None, :, :]
    h = jnp.maximum(h, 0.0)
    t = jnp.sum(h.reshape(NROW * NPAD, P) * w2_ref[...], axis=1) + b2_ref[0, 0]
    t_ref[...] = t


def _build_table(emb0, emb1, W1, b1, W2, b2):
    return pl.pallas_call(
        _table_kernel,
        out_shape=jax.ShapeDtypeStruct((NROW * NPAD,), jnp.float32),
    )(emb0, emb1, W1, b1.reshape(1, P), W2.T, b2.reshape(1, 1))


def _make_sc_gather():
    info = plsc.get_sparse_core_info()
    ncores = 1
    nw = ncores * info.num_subcores
    b_per_w = B // nw
    k_per_w = b_per_w // BLK

    mesh = plsc.VectorSubcoreMesh(core_axis_name="c", subcore_axis_name="s", num_cores=ncores)

    @functools.partial(
        pl.kernel,
        mesh=mesh,
        out_type=jax.ShapeDtypeStruct((B,), jnp.float32),
        compiler_params=pltpu.CompilerParams(
            needs_layout_passes=False, use_tc_tiling_on_sc=False
        ),
        scratch_types=[
            pltpu.VMEM((k_per_w, 2, BLK), jnp.int32),
            pltpu.VMEM((NROW * NPAD,), jnp.float32),
            pltpu.VMEM((b_per_w,), jnp.float32),
            pltpu.SemaphoreType.DMA,
            pltpu.SemaphoreType.DMA,
            pltpu.SemaphoreType.DMA,
        ],
    )
    def gather_k(xb_hbm, t_hbm, out_hbm, x_v, t_v, o_v, sem_x, sem_t, sem_o):
        wid = lax.axis_index("s") * ncores + lax.axis_index("c")
        base = wid * k_per_w
        cp_x = pltpu.async_copy(xb_hbm.at[pl.ds(base, k_per_w)], x_v, sem_x)
        cp_t = pltpu.async_copy(t_hbm, t_v, sem_t)
        cp_x.wait()
        cp_t.wait()
        out_cps = []
        for blk in range(k_per_w):
            for s in range(BLK // 16):
                x0 = x_v[blk, 0, pl.ds(s * 16, 16)]
                x1 = x_v[blk, 1, pl.ds(s * 16, 16)]
                vals = plsc.load_gather(t_v, [x0 * NPAD + x1])
                o_v[pl.ds(blk * BLK + s * 16, 16)] = vals
            out_cps.append(pltpu.async_copy(
                o_v.at[pl.ds(blk * BLK, BLK)],
                out_hbm.at[pl.ds(wid * b_per_w + blk * BLK, BLK)],
                sem_o,
            ))
        for cp in out_cps:
            cp.wait()

    return gather_k


def kernel(x, emb0, emb1, W1, b1, W2, b2):
    t = _build_table(emb0, emb1, W1, b1, W2, b2)
    xb = x.reshape(B // BLK, BLK, 2).transpose(0, 2, 1)
    out = _make_sc_gather()(xb, t)
    return out.reshape(B, 1)
